# baseline (speedup 1.0000x reference)
import functools

import jax
import jax.numpy as jnp
from jax import lax
from jax.experimental import pallas as pl
from jax.experimental.pallas import tpu as pltpu
from jax.experimental.pallas import tpu_sc as plsc

_BLK = 1000
_BIAS = 131072
_FMAX = 262143

_N = 100000
_NW = 32
_C = 3136
_NPAD = _NW * _C
_G = 56
_SUBS = _C // _G
_NROWS = _NPAD // _G
_NOUT = 100096
_TRASH = _NOUT - 1
_SLOT0 = _NOUT
_EXT = _NOUT + 2 * _NW


def _tower_body(f_ref, w1_ref, w2_ref, w3_ref, b3_ref, off_ref):
    f = f_ref[...]
    h = jnp.maximum(jnp.dot(f, w1_ref[...], preferred_element_type=jnp.float32), 0.0)
    h = jnp.maximum(jnp.dot(h, w2_ref[...], preferred_element_type=jnp.float32), 0.0)
    off_ref[...] = (
        jnp.dot(h, w3_ref[...], preferred_element_type=jnp.float32) + b3_ref[0:1, :]
    )


def _tower(feats_F, W1, W2, W3, b3):
    n = feats_F.shape[0]
    w3p = jnp.zeros((W3.shape[0], 128), jnp.float32).at[:, :3].set(W3)
    b3p = jnp.zeros((8, 128), jnp.float32).at[0, :3].set(b3)
    offp = pl.pallas_call(
        _tower_body,
        grid=(n // _BLK,),
        in_specs=[
            pl.BlockSpec((_BLK, 128), lambda i: (i, 0)),
            pl.BlockSpec((128, 64), lambda i: (0, 0)),
            pl.BlockSpec((64, 32), lambda i: (0, 0)),
            pl.BlockSpec((32, 128), lambda i: (0, 0)),
            pl.BlockSpec((8, 128), lambda i: (0, 0)),
        ],
        out_specs=pl.BlockSpec((_BLK, 128), lambda i: (i, 0)),
        out_shape=jax.ShapeDtypeStruct((n, 128), jnp.float32),
    )(feats_F, W1, W2, w3p, b3p)
    return offp


def _sc_body(
    feats,
    gidx2d,
    idx2d,
    dst2d,
    uid2d,
    keep16,
    scale16,
    score2d,
    out_ext,
    inv_ext,
    sco_ext,
    gidx_v,
    idx_v,
    dst_v,
    uid_v,
    sco_v,
    rows_v,
    keep_v,
    scale_v,
    sem,
):
    c = lax.axis_index("c")
    s = lax.axis_index("s")
    wid = s * 2 + c
    base = wid * _SUBS
    pltpu.sync_copy(gidx2d.at[pl.ds(base, _SUBS)], gidx_v)
    pltpu.sync_copy(idx2d.at[pl.ds(base, _SUBS)], idx_v)
    pltpu.sync_copy(dst2d.at[pl.ds(base, _SUBS)], dst_v)
    pltpu.sync_copy(uid2d.at[pl.ds(base, _SUBS)], uid_v)
    pltpu.sync_copy(score2d.at[pl.ds(base, _SUBS)], sco_v)

    def outer(j, acc):
        pltpu.async_copy(feats.at[gidx_v.at[j]], rows_v, sem).wait()
        off = (wid * _C + j * _G) * 16
        pltpu.sync_copy(keep16.at[pl.ds(off, _G * 16)], keep_v)
        pltpu.sync_copy(scale16.at[pl.ds(off, _G * 16)], scale_v)

        def inner(i, a):
            k = keep_v[pl.ds(i * 16, 16)]
            sc = scale_v[pl.ds(i * 16, 16)]
            out = []
            for g in range(8):
                v = rows_v[i, pl.ds(g * 16, 16)]
                nv = a[g] * k + v * sc
                rows_v[i, pl.ds(g * 16, 16)] = nv
                out.append(nv)
            return tuple(out)

        acc = lax.fori_loop(0, _G, inner, acc)
        pltpu.async_copy(rows_v, out_ext.at[dst_v.at[j]], sem).wait()
        pltpu.async_copy(uid_v.at[j], inv_ext.at[idx_v.at[j]], sem).wait()
        pltpu.async_copy(sco_v.at[j], sco_ext.at[dst_v.at[j]], sem).wait()
        return acc

    z = jnp.zeros((16,), jnp.float32)
    lax.fori_loop(0, _SUBS, outer, (z,) * 8)


@functools.partial(jax.jit, donate_argnums=())
def _sc_pooling(feats_F, gidx2d, idx2d, dst2d, uid2d, keep16, scale16, score2d):
    mesh = plsc.VectorSubcoreMesh(core_axis_name="c", subcore_axis_name="s")
    call = pl.kernel(
        _sc_body,
        mesh=mesh,
        out_type=[
            jax.ShapeDtypeStruct((_EXT, 128), jnp.float32),
            jax.ShapeDtypeStruct((_NPAD,), jnp.int32),
            jax.ShapeDtypeStruct((_EXT,), jnp.float32),
        ],
        scratch_types=[
            pltpu.VMEM((_SUBS, _G), jnp.int32),
            pltpu.VMEM((_SUBS, _G), jnp.int32),
            pltpu.VMEM((_SUBS, _G), jnp.int32),
            pltpu.VMEM((_SUBS, _G), jnp.int32),
            pltpu.VMEM((_SUBS, _G), jnp.float32),
            pltpu.VMEM((_G, 128), jnp.float32),
            pltpu.VMEM((_G * 16,), jnp.float32),
            pltpu.VMEM((_G * 16,), jnp.float32),
            pltpu.SemaphoreType.DMA,
        ],
    )
    return call(feats_F, gidx2d, idx2d, dst2d, uid2d, keep16, scale16, score2d)


def kernel(feats_F, feats_C, W1, W2, W3, b3):
    n = feats_F.shape[0]
    offp = _tower(feats_F, W1, W2, W3, b3)
    offsets = offp[:, :3]

    off_int = (jnp.sign(offsets) * jnp.expm1(jnp.abs(offsets))).astype(jnp.int32)
    new_coords = feats_C.at[:, 1:].add(off_int)

    w = new_coords[:, 0]
    xu = jnp.clip(new_coords[:, 1] + _BIAS, 0, _FMAX)
    yu = jnp.clip(new_coords[:, 2] + _BIAS, 0, _FMAX)
    zu = jnp.clip(new_coords[:, 3] + _BIAS, 0, _FMAX)
    hi = (w << 24) | (xu << 6) | (yu >> 12)
    lo = ((yu & 0xFFF) << 18) | zu

    idx = jnp.arange(n, dtype=jnp.int32)
    hi_s, lo_s, idx_s = lax.sort((hi, lo, idx), num_keys=2)

    flag = jnp.concatenate(
        [
            jnp.ones((1,), jnp.int32),
            ((hi_s[1:] != hi_s[:-1]) | (lo_s[1:] != lo_s[:-1])).astype(jnp.int32),
        ]
    )
    uid_s = jnp.cumsum(flag) - 1
    num_u = uid_s[-1] + 1

    pad = _NPAD - n
    i_p = jnp.arange(_NPAD, dtype=jnp.int32)
    flag_p = jnp.concatenate(
        [flag, jnp.ones((1,), jnp.int32), jnp.zeros((pad - 1,), jnp.int32)]
    )
    idx_p = jnp.concatenate([idx_s, jnp.arange(n, _NPAD, dtype=jnp.int32)])
    uid_p = jnp.concatenate([uid_s, jnp.full((pad,), _TRASH, jnp.int32)])

    start = lax.cummax(jnp.where(flag_p == 1, i_p, -1))
    m = jnp.flip(lax.cummin(jnp.flip(jnp.where(flag_p == 1, i_p, 2 * _NPAD))))
    end = jnp.minimum(
        jnp.concatenate([m[1:], jnp.full((1,), _NPAD, jnp.int32)]), _NPAD
    )
    cnt = (end - start).astype(jnp.float32)
    scale = jnp.where(i_p < n, 1.0 / cnt, 0.0)

    wvec = i_p // _C
    cs = wvec * _C
    ce = cs + _C
    interior = (start >= cs) & (end <= ce)
    seg_last = i_p == end - 1
    dst = jnp.where(
        interior,
        jnp.where(seg_last, uid_p, _TRASH),
        jnp.where(
            seg_last & (end <= ce),
            _SLOT0 + 2 * wvec,
            jnp.where(i_p == ce - 1, _SLOT0 + 2 * wvec + 1, _TRASH),
        ),
    ).astype(jnp.int32)

    keep16 = jnp.repeat((1 - flag_p).astype(jnp.float32), 16)
    scale16 = jnp.repeat(scale, 16)
    score = jnp.where(i_p < n, jnp.log1p(cnt), 0.0)

    gidx_p = jnp.where(i_p < n, jnp.concatenate([idx_s, jnp.zeros((pad,), jnp.int32)]), 0)
    feats_ext, inverse_ext, scores_ext = _sc_pooling(
        feats_F,
        gidx_p.reshape(_NROWS, _G),
        idx_p.reshape(_NROWS, _G),
        dst.reshape(_NROWS, _G),
        uid_p.reshape(_NROWS, _G),
        keep16,
        scale16,
        score.reshape(_NROWS, _G),
    )

    slot_uid = (
        jnp.full((2 * _NW,), _TRASH, jnp.int32)
        .at[dst - _SLOT0]
        .set(uid_p, mode="drop")
    )
    rid = jnp.arange(_NOUT, dtype=jnp.int32)
    out_feats = jnp.where((rid < num_u)[:, None], feats_ext[:_NOUT], 0.0)
    out_feats = out_feats.at[slot_uid].set(0.0, mode="drop")
    out_feats = out_feats.at[slot_uid].add(feats_ext[_SLOT0:], mode="drop")[:n]

    out_scores = jnp.where(rid < num_u, scores_ext[:_NOUT], 0.0)
    out_scores = out_scores.at[slot_uid].set(scores_ext[_SLOT0:], mode="drop")
    out_scores = out_scores[:n, None]

    inverse = inverse_ext[:n]
    out_coords = (
        jnp.zeros((n, 4), jnp.int32).at[uid_s].set(new_coords[idx_s], mode="drop")
    )
    return (offsets, out_coords, out_feats, out_scores, inverse.astype(jnp.int64))
